# baseline (speedup 1.0000x reference)
import functools

import jax
import jax.numpy as jnp
from jax import lax
from jax.experimental import pallas as pl
from jax.experimental.pallas import tpu as pltpu
from jax.experimental.pallas import tpu_sc as plsc

_N = 100000
_E = 3200000
NP = 106496
EP = 3211264
SUB = 128
ROWS = 4
EB = ROWS * SUB
EPR = EP // SUB
NBLKG = EPR // ROWS + 2
RPT = NP // 16
ZR = 512
RBL = 8192
NBLK_TC = NP // RBL

_mesh = plsc.VectorSubcoreMesh(core_axis_name="c", subcore_axis_name="s")


def _edge_pass(F, mode):
    shape = (NP,) if F == 0 else (NP, F)
    rows_shape = (EB,) if F == 0 else (EB, F)
    nblk = EP // EB // (16 if mode == "split" else 32)
    nj = nblk // 2

    IR = ROWS if mode == "deg" else 2 * ROWS

    @functools.partial(
        pl.kernel,
        out_type=jax.ShapeDtypeStruct((2,) + shape, jnp.float32),
        mesh=_mesh,
        compiler_params=pltpu.CompilerParams(use_tc_tiling_on_sc=False),
        scratch_types=[
            pltpu.VMEM((IR, SUB), jnp.int32),
            pltpu.VMEM((IR, SUB), jnp.int32),
            pltpu.VMEM(rows_shape, jnp.float32),
            pltpu.VMEM(rows_shape, jnp.float32),
            pltpu.VMEM_SHARED(shape, jnp.float32),
            pltpu.SemaphoreType.DMA,
            pltpu.SemaphoreType.DMA,
            pltpu.SemaphoreType.DMA,
            pltpu.SemaphoreType.DMA,
            pltpu.SemaphoreType.DMA,
            pltpu.SemaphoreType.DMA,
        ],
    )
    def k(*refs):
        if mode == "deg":
            (cidx, out, cA, cB, rowsA, rowsB, acc_sh,
             gsA, gsB, isA, isB, ssA, ssB) = refs
            g = None
        else:
            (g, cidx, out, cA, cB, rowsA, rowsB,
             acc_sh, gsA, gsB, isA, isB, ssA, ssB) = refs
        c = lax.axis_index("c")
        s = lax.axis_index("s")
        r0 = s * RPT

        def zfill(rows_v, val):
            vec = jnp.full((16,), val, jnp.float32)
            n16 = (EB // 16) if F == 0 else EB * F // 16

            def fz(i, _):
                if F == 0:
                    rows_v[pl.ds(i * 16, 16)] = vec
                else:
                    r = i // (F // 16)
                    q = i % (F // 16)
                    rows_v[r, pl.ds(q * 16, 16)] = vec
                return 0
            lax.fori_loop(0, n16, fz, 0)

        def zinit():
            zsrc = rowsA.at[pl.ds(0, ZR)]
            for i in range(RPT // ZR):
                pltpu.sync_copy(zsrc, acc_sh.at[pl.ds(r0 + i * ZR, ZR)])

        if mode == "split":
            pltpu.sync_copy(g.at[c].at[pl.ds(r0, RPT)], acc_sh.at[pl.ds(r0, RPT)])
        elif mode == "dup":
            zfill(rowsA, 0.0)

            @pl.when(c == 0)
            def _():
                pltpu.sync_copy(g.at[pl.ds(r0, RPT)], acc_sh.at[pl.ds(r0, RPT)])

            @pl.when(c == 1)
            def _():
                zinit()
        else:
            zfill(rowsA, 0.0)
            zinit()
            zfill(rowsA, 1.0)

        plsc.subcore_barrier()

        if mode == "split":
            tile_blk0 = s * nblk
        else:
            tile_blk0 = (c * 16 + s) * nblk

        def firei(blk, cbuf, sem):
            pltpu.async_copy(cidx.at[pl.ds((tile_blk0 + blk) * IR, IR)],
                             cbuf, sem)

        def waiti(cbuf, sem):
            pltpu.make_async_copy(cidx.at[pl.ds(0, IR)], cbuf, sem).wait()

        def sdrain(sem):
            for k2 in range(ROWS):
                pltpu.make_async_copy(rowsA.at[pl.ds(k2 * SUB, SUB)],
                                      acc_sh.at[pl.ds(0, SUB)], sem).wait()

        if mode == "deg":
            def scat1(cbuf, sem):
                for k2 in range(ROWS):
                    pltpu.async_copy(rowsA.at[pl.ds(k2 * SUB, SUB)],
                                     acc_sh.at[cbuf.at[k2]], sem, add=True)

            firei(0, cA, isA)
            waiti(cA, isA)
            firei(1, cB, isB)

            def body(j, _):
                waiti(cB, isB)
                scat1(cA, ssA)
                scat1(cB, ssB)
                sdrain(ssA)
                firei(2 * j + 2, cA, isA)
                sdrain(ssB)
                waiti(cA, isA)
                firei(2 * j + 3, cB, isB)
                return 0

            lax.fori_loop(0, nj, body, 0)
            waiti(cB, isB)
        else:
            gsrc = (lambda: g.at[c]) if mode == "split" else (lambda: g)
            glin = g.at[0] if mode == "split" else g

            def fire(cbuf, rows_v, sem):
                for k2 in range(ROWS):
                    pltpu.async_copy(gsrc().at[cbuf.at[k2]],
                                     rows_v.at[pl.ds(k2 * SUB, SUB)], sem)

            def drain(rows_v, sem):
                for k2 in range(ROWS):
                    pltpu.make_async_copy(glin.at[pl.ds(0, SUB)],
                                          rows_v.at[pl.ds(k2 * SUB, SUB)],
                                          sem).wait()

            def scat(rows_v, cbuf, sem):
                for k2 in range(ROWS):
                    pltpu.async_copy(rows_v.at[pl.ds(k2 * SUB, SUB)],
                                     acc_sh.at[cbuf.at[ROWS + k2]], sem,
                                     add=True)

            firei(0, cA, isA)
            waiti(cA, isA)
            fire(cA, rowsA, gsA)
            firei(1, cB, isB)

            def body(j, _):
                waiti(cB, isB)
                fire(cB, rowsB, gsB)
                drain(rowsA, gsA)
                scat(rowsA, cA, ssA)
                drain(rowsB, gsB)
                scat(rowsB, cB, ssB)
                sdrain(ssA)
                firei(2 * j + 2, cA, isA)
                waiti(cA, isA)
                fire(cA, rowsA, gsA)
                sdrain(ssB)
                firei(2 * j + 3, cB, isB)
                return 0

            lax.fori_loop(0, nj, body, 0)
            drain(rowsA, gsA)
            waiti(cB, isB)

        plsc.subcore_barrier()
        pltpu.sync_copy(acc_sh.at[pl.ds(r0, RPT)], out.at[c].at[pl.ds(r0, RPT)])

    return k


_deg_pass = _edge_pass(0, "deg")
_l1_pass = _edge_pass(16, "dup")
_l2_pass = _edge_pass(16, "split")
_l3_pass = _edge_pass(0, "dup")


def _tc0(degp_ref, xT_ref, disT_ref, t1T_ref):
    d = 1.0 + degp_ref[0] + degp_ref[1]
    dis = lax.rsqrt(d)
    disT_ref[...] = dis[None]
    t1T_ref[...] = jnp.concatenate(
        [dis[None] * xT_ref[...], jnp.zeros((12, RBL), jnp.float32)], axis=0)


def _tc1(a1T_ref, disT_ref, w1t_ref, b1c_ref, t2sT_ref):
    dis = disT_ref[...]
    out1 = (dis * (a1T_ref[0] + a1T_ref[1]))[:4]
    h = jnp.maximum(jnp.dot(w1t_ref[...], out1,
                            preferred_element_type=jnp.float32) + b1c_ref[...], 0.0)
    t2 = dis * h
    t2sT_ref[0] = t2[:16]
    t2sT_ref[1] = t2[16:]


def _tc2(a2T_ref, disT_ref, w2t_ref, b2c_ref, w3t_ref, t3T_ref):
    dis = disT_ref[...]
    out2 = dis * jnp.concatenate([a2T_ref[0], a2T_ref[1]], axis=0)
    h = jnp.maximum(jnp.dot(w2t_ref[...], out2,
                            preferred_element_type=jnp.float32) + b2c_ref[...], 0.0)
    t3T_ref[...] = dis * jnp.dot(w3t_ref[...], h,
                                 preferred_element_type=jnp.float32)


def _tc3(a3_ref, disT_ref, b3_ref, yT_ref):
    yT_ref[...] = disT_ref[...] * (a3_ref[0] + a3_ref[1])[None] + b3_ref[...]


def _lanespec(F):
    return pl.BlockSpec((F, RBL), lambda i: (0, i))


def _stacklanespec(F):
    return pl.BlockSpec((2, F, RBL), lambda i: (0, 0, i))


def _pairspec():
    return pl.BlockSpec((2, RBL), lambda i: (0, i))


def _fullspec(a, b):
    return pl.BlockSpec((a, b), lambda i: (0, 0))


def kernel(x, edge_index, W1, b1, W2, b2, W3, b3):
    f32 = jnp.float32
    src = edge_index[0].astype(jnp.int32)
    dst = edge_index[1].astype(jnp.int32)
    pad = _N + (jnp.arange(EP - _E, dtype=jnp.int32) % (NP - _N))
    taile = jnp.arange(2 * EB, dtype=jnp.int32) % _N
    sa = jnp.concatenate([src, pad, taile]).reshape(NBLKG, ROWS, SUB)
    da = jnp.concatenate([dst, pad, taile]).reshape(NBLKG, ROWS, SUB)
    cidx = jnp.concatenate([sa, da], axis=1).reshape(NBLKG * 2 * ROWS, SUB)
    dcidx = da.reshape(NBLKG * ROWS, SUB)

    xT_pad = jnp.concatenate([x.T, jnp.zeros((4, NP - _N), f32)], axis=1)

    degp = _deg_pass(dcidx)

    disT, t1T = pl.pallas_call(
        _tc0,
        grid=(NBLK_TC,),
        in_specs=[_pairspec(), _lanespec(4)],
        out_specs=[_lanespec(1), _lanespec(16)],
        out_shape=[jax.ShapeDtypeStruct((1, NP), f32),
                   jax.ShapeDtypeStruct((16, NP), f32)],
    )(degp, xT_pad)

    a1 = _l1_pass(t1T.T, cidx)

    t2sT = pl.pallas_call(
        _tc1,
        grid=(NBLK_TC,),
        in_specs=[_stacklanespec(16), _lanespec(1), _fullspec(32, 4),
                  _fullspec(32, 1)],
        out_specs=_stacklanespec(16),
        out_shape=jax.ShapeDtypeStruct((2, 16, NP), f32),
    )(jnp.transpose(a1, (0, 2, 1)), disT, W1.T, b1.reshape(32, 1))

    a2 = _l2_pass(jnp.transpose(t2sT, (0, 2, 1)), cidx)

    t3T = pl.pallas_call(
        _tc2,
        grid=(NBLK_TC,),
        in_specs=[_stacklanespec(16), _lanespec(1), _fullspec(32, 32),
                  _fullspec(32, 1), _fullspec(1, 32)],
        out_specs=_lanespec(1),
        out_shape=jax.ShapeDtypeStruct((1, NP), f32),
    )(jnp.transpose(a2, (0, 2, 1)), disT, W2.T, b2.reshape(32, 1), W3.T)

    a3 = _l3_pass(t3T.reshape(NP), cidx)

    yT = pl.pallas_call(
        _tc3,
        grid=(NBLK_TC,),
        in_specs=[_pairspec(), _lanespec(1), _fullspec(1, 1)],
        out_specs=_lanespec(1),
        out_shape=jax.ShapeDtypeStruct((1, NP), f32),
    )(a3, disT, b3.reshape(1, 1))

    return yT.reshape(NP, 1)[:_N]


# speedup vs baseline: 1.0135x; 1.0135x over previous
import functools

import jax
import jax.numpy as jnp
from jax import lax
from jax.experimental import pallas as pl
from jax.experimental.pallas import tpu as pltpu
from jax.experimental.pallas import tpu_sc as plsc

_N = 100000
_E = 3200000
NP = 106496
EP = 3211264
SUB = 128
ROWS = 4
EB = ROWS * SUB
EPR = EP // SUB
NBLKG = EPR // ROWS + 2
RPT = NP // 16
ZR = 512
RBL = 8192
NBLK_TC = NP // RBL

_mesh = plsc.VectorSubcoreMesh(core_axis_name="c", subcore_axis_name="s")


def _edge_pass(F, mode):
    shape = (NP,) if F == 0 else (NP, F)
    rows_shape = (EB,) if F == 0 else (EB, F)
    nblk = EP // EB // (16 if mode == "split" else 32)
    nj = nblk // 2

    IR = ROWS if mode == "deg" else 2 * ROWS

    @functools.partial(
        pl.kernel,
        out_type=jax.ShapeDtypeStruct((2,) + shape, jnp.float32),
        mesh=_mesh,
        compiler_params=pltpu.CompilerParams(use_tc_tiling_on_sc=False),
        scratch_types=[
            pltpu.VMEM((IR, SUB), jnp.int32),
            pltpu.VMEM((IR, SUB), jnp.int32),
            pltpu.VMEM(rows_shape, jnp.float32),
            pltpu.VMEM(rows_shape, jnp.float32),
            pltpu.VMEM_SHARED(shape, jnp.float32),
            pltpu.SemaphoreType.DMA,
            pltpu.SemaphoreType.DMA,
            pltpu.SemaphoreType.DMA,
            pltpu.SemaphoreType.DMA,
        ],
    )
    def k(*refs):
        if mode == "deg":
            (cidx, out, cA, cB, rowsA, rowsB, acc_sh,
             gsA, gsB, isA, isB) = refs
            g = None
        else:
            (g, cidx, out, cA, cB, rowsA, rowsB,
             acc_sh, gsA, gsB, isA, isB) = refs
        c = lax.axis_index("c")
        s = lax.axis_index("s")
        r0 = s * RPT

        def zfill(rows_v, val):
            vec = jnp.full((16,), val, jnp.float32)
            n16 = (EB // 16) if F == 0 else EB * F // 16

            def fz(i, _):
                if F == 0:
                    rows_v[pl.ds(i * 16, 16)] = vec
                else:
                    r = i // (F // 16)
                    q = i % (F // 16)
                    rows_v[r, pl.ds(q * 16, 16)] = vec
                return 0
            lax.fori_loop(0, n16, fz, 0)

        def zinit():
            zsrc = rowsA.at[pl.ds(0, ZR)]
            for i in range(RPT // ZR):
                pltpu.sync_copy(zsrc, acc_sh.at[pl.ds(r0 + i * ZR, ZR)])

        if mode == "split":
            pltpu.sync_copy(g.at[c].at[pl.ds(r0, RPT)], acc_sh.at[pl.ds(r0, RPT)])
        elif mode == "dup":
            zfill(rowsA, 0.0)

            @pl.when(c == 0)
            def _():
                pltpu.sync_copy(g.at[pl.ds(r0, RPT)], acc_sh.at[pl.ds(r0, RPT)])

            @pl.when(c == 1)
            def _():
                zinit()
        else:
            zfill(rowsA, 0.0)
            zinit()
            zfill(rowsA, 1.0)

        plsc.subcore_barrier()

        if mode == "split":
            tile_blk0 = s * nblk
        else:
            tile_blk0 = (c * 16 + s) * nblk

        def firei(blk, cbuf, sem):
            pltpu.async_copy(cidx.at[pl.ds((tile_blk0 + blk) * IR, IR)],
                             cbuf, sem)

        def waiti(cbuf, sem):
            pltpu.make_async_copy(cidx.at[pl.ds(0, IR)], cbuf, sem).wait()

        if mode == "deg":
            def scat1(cbuf):
                for k2 in range(ROWS):
                    pltpu.sync_copy(rowsA.at[pl.ds(k2 * SUB, SUB)],
                                    acc_sh.at[cbuf.at[k2]], add=True)

            firei(0, cA, isA)
            waiti(cA, isA)
            firei(1, cB, isB)

            def body(j, _):
                waiti(cB, isB)
                scat1(cA)
                firei(2 * j + 2, cA, isA)
                scat1(cB)
                waiti(cA, isA)
                firei(2 * j + 3, cB, isB)
                return 0

            lax.fori_loop(0, nj, body, 0)
            waiti(cB, isB)
        else:
            gsrc = (lambda: g.at[c]) if mode == "split" else (lambda: g)
            glin = g.at[0] if mode == "split" else g

            def fire(cbuf, rows_v, sem):
                for k2 in range(ROWS):
                    pltpu.async_copy(gsrc().at[cbuf.at[k2]],
                                     rows_v.at[pl.ds(k2 * SUB, SUB)], sem)

            def drain(rows_v, sem):
                for k2 in range(ROWS):
                    pltpu.make_async_copy(glin.at[pl.ds(0, SUB)],
                                          rows_v.at[pl.ds(k2 * SUB, SUB)],
                                          sem).wait()

            def scat(rows_v, cbuf):
                for k2 in range(ROWS):
                    pltpu.sync_copy(rows_v.at[pl.ds(k2 * SUB, SUB)],
                                    acc_sh.at[cbuf.at[ROWS + k2]], add=True)

            firei(0, cA, isA)
            waiti(cA, isA)
            fire(cA, rowsA, gsA)
            firei(1, cB, isB)

            def body(j, _):
                waiti(cB, isB)
                fire(cB, rowsB, gsB)
                drain(rowsA, gsA)
                scat(rowsA, cA)
                firei(2 * j + 2, cA, isA)
                drain(rowsB, gsB)
                scat(rowsB, cB)
                waiti(cA, isA)
                fire(cA, rowsA, gsA)
                firei(2 * j + 3, cB, isB)
                return 0

            lax.fori_loop(0, nj, body, 0)
            drain(rowsA, gsA)
            waiti(cB, isB)

        plsc.subcore_barrier()
        pltpu.sync_copy(acc_sh.at[pl.ds(r0, RPT)], out.at[c].at[pl.ds(r0, RPT)])

    return k


_deg_pass = _edge_pass(0, "deg")
_l1_pass = _edge_pass(16, "dup")
_l2_pass = _edge_pass(16, "split")
_l3_pass = _edge_pass(0, "dup")


def _tc0(degp_ref, xT_ref, disT_ref, t1T_ref):
    d = 1.0 + degp_ref[0] + degp_ref[1]
    dis = lax.rsqrt(d)
    disT_ref[...] = dis[None]
    t1T_ref[...] = jnp.concatenate(
        [dis[None] * xT_ref[...], jnp.zeros((12, RBL), jnp.float32)], axis=0)


def _tc1(a1T_ref, disT_ref, w1t_ref, b1c_ref, t2sT_ref):
    dis = disT_ref[...]
    out1 = (dis * (a1T_ref[0] + a1T_ref[1]))[:4]
    h = jnp.maximum(jnp.dot(w1t_ref[...], out1,
                            preferred_element_type=jnp.float32) + b1c_ref[...], 0.0)
    t2 = dis * h
    t2sT_ref[0] = t2[:16]
    t2sT_ref[1] = t2[16:]


def _tc2(a2T_ref, disT_ref, w2t_ref, b2c_ref, w3t_ref, t3T_ref):
    dis = disT_ref[...]
    out2 = dis * jnp.concatenate([a2T_ref[0], a2T_ref[1]], axis=0)
    h = jnp.maximum(jnp.dot(w2t_ref[...], out2,
                            preferred_element_type=jnp.float32) + b2c_ref[...], 0.0)
    t3T_ref[...] = dis * jnp.dot(w3t_ref[...], h,
                                 preferred_element_type=jnp.float32)


def _tc3(a3_ref, disT_ref, b3_ref, yT_ref):
    yT_ref[...] = disT_ref[...] * (a3_ref[0] + a3_ref[1])[None] + b3_ref[...]


def _lanespec(F):
    return pl.BlockSpec((F, RBL), lambda i: (0, i))


def _stacklanespec(F):
    return pl.BlockSpec((2, F, RBL), lambda i: (0, 0, i))


def _pairspec():
    return pl.BlockSpec((2, RBL), lambda i: (0, i))


def _fullspec(a, b):
    return pl.BlockSpec((a, b), lambda i: (0, 0))


def kernel(x, edge_index, W1, b1, W2, b2, W3, b3):
    f32 = jnp.float32
    src = edge_index[0].astype(jnp.int32)
    dst = edge_index[1].astype(jnp.int32)
    pad = _N + (jnp.arange(EP - _E, dtype=jnp.int32) % (NP - _N))
    taile = jnp.arange(2 * EB, dtype=jnp.int32) % _N
    sa = jnp.concatenate([src, pad, taile]).reshape(NBLKG, ROWS, SUB)
    da = jnp.concatenate([dst, pad, taile]).reshape(NBLKG, ROWS, SUB)
    cidx = jnp.concatenate([sa, da], axis=1).reshape(NBLKG * 2 * ROWS, SUB)
    dcidx = da.reshape(NBLKG * ROWS, SUB)

    xT_pad = jnp.concatenate([x.T, jnp.zeros((4, NP - _N), f32)], axis=1)

    degp = _deg_pass(dcidx)

    disT, t1T = pl.pallas_call(
        _tc0,
        grid=(NBLK_TC,),
        in_specs=[_pairspec(), _lanespec(4)],
        out_specs=[_lanespec(1), _lanespec(16)],
        out_shape=[jax.ShapeDtypeStruct((1, NP), f32),
                   jax.ShapeDtypeStruct((16, NP), f32)],
    )(degp, xT_pad)

    a1 = _l1_pass(t1T.T, cidx)

    t2sT = pl.pallas_call(
        _tc1,
        grid=(NBLK_TC,),
        in_specs=[_stacklanespec(16), _lanespec(1), _fullspec(32, 4),
                  _fullspec(32, 1)],
        out_specs=_stacklanespec(16),
        out_shape=jax.ShapeDtypeStruct((2, 16, NP), f32),
    )(jnp.transpose(a1, (0, 2, 1)), disT, W1.T, b1.reshape(32, 1))

    a2 = _l2_pass(jnp.transpose(t2sT, (0, 2, 1)), cidx)

    t3T = pl.pallas_call(
        _tc2,
        grid=(NBLK_TC,),
        in_specs=[_stacklanespec(16), _lanespec(1), _fullspec(32, 32),
                  _fullspec(32, 1), _fullspec(1, 32)],
        out_specs=_lanespec(1),
        out_shape=jax.ShapeDtypeStruct((1, NP), f32),
    )(jnp.transpose(a2, (0, 2, 1)), disT, W2.T, b2.reshape(32, 1), W3.T)

    a3 = _l3_pass(t3T.reshape(NP), cidx)

    yT = pl.pallas_call(
        _tc3,
        grid=(NBLK_TC,),
        in_specs=[_pairspec(), _lanespec(1), _fullspec(1, 1)],
        out_specs=_lanespec(1),
        out_shape=jax.ShapeDtypeStruct((1, NP), f32),
    )(a3, disT, b3.reshape(1, 1))

    return yT.reshape(NP, 1)[:_N]
